# baseline (speedup 1.0000x reference)
import jax
import jax.numpy as jnp
from jax import lax
from jax.experimental import pallas as pl
from jax.experimental.pallas import tpu as pltpu
from jax.experimental.pallas import tpu_sc as plsc

_B, _C, _H, _W = 4, 96, 512, 512
_HW = _H * _W
_NC, _NS = 2, 16
_NW = _NC * _NS
_CHUNK = _HW // _NW
_NPLANES = _B * _C
_LANES = 16
_UNROLL = 8
_STEPS = _CHUNK // (_LANES * _UNROLL)


def _sc_partials_body(img, mask, out_s, out_ss, out_n,
                      mask_v, buf, s_v, ss_v, n_v, sem0, sem1):
    wid = lax.axis_index("s") * _NC + lax.axis_index("c")
    off = wid * _CHUNK
    sems = (sem0, sem1)
    zero = jnp.zeros((_LANES,), jnp.float32)

    for b in range(_B):
        pltpu.sync_copy(mask.at[b, pl.ds(off, _CHUNK)],
                        mask_v.at[pl.ds(b * _CHUNK, _CHUNK)])

    def issue(b, c, slot):
        pltpu.async_copy(img.at[b, c, pl.ds(off, _CHUNK)], buf.at[slot],
                         sems[slot])

    issue(0, 0, 0)
    issue(1, 0, 1)

    def n_body(i, acc):
        return acc + mask_v[pl.ds(i * _LANES, _LANES)]
    n_v[...] = lax.fori_loop(0, (_B * _CHUNK) // _LANES, n_body, zero)

    def channel(c, carry):
        del carry
        s_vec = zero
        ss_vec = zero
        for b in range(_B):
            slot = b % 2
            pltpu.make_async_copy(img.at[0, 0, pl.ds(0, _CHUNK)],
                                  buf.at[slot], sems[slot]).wait()
            mbase = b * _CHUNK

            def step(i, acc):
                sv, ssv = acc
                base = i * (_LANES * _UNROLL)
                for u in range(_UNROLL):
                    x = buf[slot, pl.ds(base + u * _LANES, _LANES)]
                    m = mask_v[pl.ds(mbase + base + u * _LANES, _LANES)]
                    xm = x * m
                    sv = sv + xm
                    ssv = ssv + xm * xm
                return (sv, ssv)

            s_vec, ss_vec = lax.fori_loop(0, _STEPS, step, (s_vec, ss_vec))

            nb = (b + 2) % _B
            nc = c + (b + 2) // _B
            if b < 2:
                issue(nb, nc, slot)
            else:
                @pl.when(nc < _C)
                def _():
                    issue(nb, nc, slot)

        s_v[c] = s_vec
        ss_v[c] = ss_vec
        return 0

    lax.fori_loop(0, _C, channel, 0)

    pltpu.sync_copy(s_v, out_s.at[wid])
    pltpu.sync_copy(ss_v, out_ss.at[wid])
    pltpu.sync_copy(n_v, out_n.at[wid])


def _finalize_body(s_ref, ss_ref, n_ref, out_ref):
    s = jnp.sum(s_ref[...], axis=(0, 2))
    ss = jnp.sum(ss_ref[...], axis=(0, 2))
    n = jnp.sum(n_ref[...])
    var = (ss - s * s / n) / (n - 1.0)
    std = jnp.sqrt(var)
    out_ref[...] = (jnp.sum(std) / _C).reshape(1, 1)


@jax.jit
def kernel(img, mask):
    img3 = img.reshape(_B, _C, _HW)
    mask_f = mask.reshape(_B, _HW).astype(jnp.float32)

    mesh = plsc.VectorSubcoreMesh(core_axis_name="c", subcore_axis_name="s")
    sc_partials = pl.kernel(
        _sc_partials_body,
        out_type=(
            jax.ShapeDtypeStruct((_NW, _C, _LANES), jnp.float32),
            jax.ShapeDtypeStruct((_NW, _C, _LANES), jnp.float32),
            jax.ShapeDtypeStruct((_NW, _LANES), jnp.float32),
        ),
        mesh=mesh,
        scratch_types=[
            pltpu.VMEM((_B * _CHUNK,), jnp.float32),
            pltpu.VMEM((2, _CHUNK), jnp.float32),
            pltpu.VMEM((_C, _LANES), jnp.float32),
            pltpu.VMEM((_C, _LANES), jnp.float32),
            pltpu.VMEM((_LANES,), jnp.float32),
            pltpu.SemaphoreType.DMA,
            pltpu.SemaphoreType.DMA,
        ],
    )
    part_s, part_ss, part_n = sc_partials(img3, mask_f)

    loss = pl.pallas_call(
        _finalize_body,
        out_shape=jax.ShapeDtypeStruct((1, 1), jnp.float32),
    )(part_s, part_ss, part_n)
    return loss[0, 0]


# speedup vs baseline: 2.5579x; 2.5579x over previous
import jax
import jax.numpy as jnp
from jax import lax
from jax.experimental import pallas as pl
from jax.experimental.pallas import tpu as pltpu
from jax.experimental.pallas import tpu_sc as plsc

_B, _C, _H, _W = 4, 96, 512, 512
_HW = _H * _W
_NC, _NS = 2, 16
_NW = _NC * _NS
_ROWS = _H // _NW
_CHUNK = _ROWS * _W
_NPLANES = _B * _C
_LANES = 16
_UNROLL = 8
_STEPS = _CHUNK // (_LANES * _UNROLL)
_VPR = _W // _LANES
_SPR = _VPR // _UNROLL


def _sc_partials_body(img, mask, out_s, out_ss, out_n,
                      mask_v, buf, s_v, ss_v, n_v, sem0, sem1):
    wid = lax.axis_index("s") * _NC + lax.axis_index("c")
    off = wid * _CHUNK
    row0 = wid * _ROWS
    sems = (sem0, sem1)
    zero = jnp.zeros((_LANES,), jnp.float32)

    for b in range(_B):
        pltpu.sync_copy(mask.at[b, pl.ds(off, _CHUNK)],
                        mask_v.at[pl.ds(b * _CHUNK, _CHUNK)])

    def issue(b, c, slot):
        pltpu.async_copy(img.at[b, c, pl.ds(row0, _ROWS), :], buf.at[slot],
                         sems[slot])

    issue(0, 0, 0)
    issue(1, 0, 1)

    def n_body(i, acc):
        return acc + mask_v[pl.ds(i * _LANES, _LANES)]
    n_v[...] = lax.fori_loop(0, (_B * _CHUNK) // _LANES, n_body, zero)

    def channel(c, carry):
        del carry
        s_vec = zero
        ss_vec = zero
        for b in range(_B):
            slot = b % 2
            pltpu.make_async_copy(img.at[0, 0, pl.ds(0, _ROWS), :],
                                  buf.at[slot], sems[slot]).wait()
            mbase = b * _CHUNK

            def step(i, acc):
                sv, ssv = acc
                r = lax.div(i, _SPR)
                cbase = lax.rem(i, _SPR) * (_LANES * _UNROLL)
                fbase = mbase + i * (_LANES * _UNROLL)
                for u in range(_UNROLL):
                    x = buf[slot, r, pl.ds(cbase + u * _LANES, _LANES)]
                    m = mask_v[pl.ds(fbase + u * _LANES, _LANES)]
                    xm = x * m
                    sv = sv + xm
                    ssv = ssv + xm * xm
                return (sv, ssv)

            s_vec, ss_vec = lax.fori_loop(0, _STEPS, step, (s_vec, ss_vec))

            nb = (b + 2) % _B
            nc = c + (b + 2) // _B
            if b < 2:
                issue(nb, nc, slot)
            else:
                @pl.when(nc < _C)
                def _():
                    issue(nb, nc, slot)

        s_v[c] = s_vec
        ss_v[c] = ss_vec
        return 0

    lax.fori_loop(0, _C, channel, 0)

    pltpu.sync_copy(s_v, out_s.at[wid])
    pltpu.sync_copy(ss_v, out_ss.at[wid])
    pltpu.sync_copy(n_v, out_n.at[wid])


def _finalize_body(s_ref, ss_ref, n_ref, out_ref):
    s = jnp.sum(s_ref[...], axis=(0, 2))
    ss = jnp.sum(ss_ref[...], axis=(0, 2))
    n = jnp.sum(n_ref[...])
    var = (ss - s * s / n) / (n - 1.0)
    std = jnp.sqrt(var)
    out_ref[...] = (jnp.sum(std) / _C).reshape(1, 1)


@jax.jit
def kernel(img, mask):
    mask_f = mask.reshape(_B, _HW).astype(jnp.float32)

    mesh = plsc.VectorSubcoreMesh(core_axis_name="c", subcore_axis_name="s")
    sc_partials = pl.kernel(
        _sc_partials_body,
        out_type=(
            jax.ShapeDtypeStruct((_NW, _C, _LANES), jnp.float32),
            jax.ShapeDtypeStruct((_NW, _C, _LANES), jnp.float32),
            jax.ShapeDtypeStruct((_NW, _LANES), jnp.float32),
        ),
        mesh=mesh,
        scratch_types=[
            pltpu.VMEM((_B * _CHUNK,), jnp.float32),
            pltpu.VMEM((2, _ROWS, _W), jnp.float32),
            pltpu.VMEM((_C, _LANES), jnp.float32),
            pltpu.VMEM((_C, _LANES), jnp.float32),
            pltpu.VMEM((_LANES,), jnp.float32),
            pltpu.SemaphoreType.DMA,
            pltpu.SemaphoreType.DMA,
        ],
    )
    part_s, part_ss, part_n = sc_partials(img, mask_f)

    loss = pl.pallas_call(
        _finalize_body,
        out_shape=jax.ShapeDtypeStruct((1, 1), jnp.float32),
    )(part_s, part_ss, part_n)
    return loss[0, 0]


# speedup vs baseline: 4.0160x; 1.5700x over previous
import jax
import jax.numpy as jnp
from jax import lax
from jax.experimental import pallas as pl
from jax.experimental.pallas import tpu as pltpu
from jax.experimental.pallas import tpu_sc as plsc

_B, _C, _H, _W = 4, 96, 512, 512
_HW = _H * _W
_NC, _NS = 2, 16
_NW = _NC * _NS
_ROWS = _H // _NW
_CHUNK = _ROWS * _W
_LANES = 16
_G = 4
_NG = _C // _G
_UNROLL = 4
_STEPS = _CHUNK // (_LANES * _UNROLL)
_VPR = _W // _LANES


def _sc_partials_body(img, mask, out_s, out_ss, out_n,
                      mask_v, buf, s_v, ss_v, n_v, sem0, sem1):
    wid = lax.axis_index("s") * _NC + lax.axis_index("c")
    off = wid * _CHUNK
    row0 = wid * _ROWS
    sems = (sem0, sem1)
    zero = jnp.zeros((_LANES,), jnp.float32)

    for b in range(_B):
        pltpu.sync_copy(mask.at[b, pl.ds(off, _CHUNK)],
                        mask_v.at[pl.ds(b * _CHUNK, _CHUNK)])

    def issue(b, c0, slot):
        pltpu.async_copy(
            img.at[b, pl.ds(c0, _G), pl.ds(row0, _ROWS), :],
            buf.at[slot], sems[slot])

    issue(0, 0, 0)
    issue(1, 0, 1)

    def n_body(i, acc):
        return acc + mask_v[pl.ds(i * _LANES, _LANES)]
    n_v[...] = lax.fori_loop(0, (_B * _CHUNK) // _LANES, n_body, zero)

    def group(gi, carry):
        del carry
        c0 = gi * _G
        acc = [zero] * (2 * _G)
        for b in range(_B):
            slot = b % 2
            pltpu.make_async_copy(
                img.at[0, pl.ds(0, _G), pl.ds(0, _ROWS), :],
                buf.at[slot], sems[slot]).wait()
            mbase = b * _CHUNK

            def step(i, a):
                a = list(a)
                r = lax.div(i * _UNROLL, _VPR)
                cb = lax.rem(i * _UNROLL, _VPR) * _LANES
                fb = mbase + i * (_LANES * _UNROLL)
                for u in range(_UNROLL):
                    m = mask_v[pl.ds(fb + u * _LANES, _LANES)]
                    for g in range(_G):
                        x = buf[slot, g, r, pl.ds(cb + u * _LANES, _LANES)]
                        xm = x * m
                        a[2 * g] = a[2 * g] + xm
                        a[2 * g + 1] = a[2 * g + 1] + xm * xm
                return tuple(a)

            acc = list(lax.fori_loop(0, _STEPS, step, tuple(acc)))

            nb = (b + 2) % _B
            ngi = gi + (b + 2) // _B
            if b < 2:
                issue(nb, c0, slot)
            else:
                @pl.when(ngi < _NG)
                def _():
                    issue(nb, ngi * _G, slot)

        for g in range(_G):
            s_v[c0 + g] = acc[2 * g]
            ss_v[c0 + g] = acc[2 * g + 1]
        return 0

    lax.fori_loop(0, _NG, group, 0)

    pltpu.sync_copy(s_v, out_s.at[wid])
    pltpu.sync_copy(ss_v, out_ss.at[wid])
    pltpu.sync_copy(n_v, out_n.at[wid])


def _finalize_body(s_ref, ss_ref, n_ref, out_ref):
    s = jnp.sum(s_ref[...], axis=(0, 2))
    ss = jnp.sum(ss_ref[...], axis=(0, 2))
    n = jnp.sum(n_ref[...])
    var = (ss - s * s / n) / (n - 1.0)
    std = jnp.sqrt(var)
    out_ref[...] = (jnp.sum(std) / _C).reshape(1, 1)


@jax.jit
def kernel(img, mask):
    mask_f = mask.reshape(_B, _HW).astype(jnp.float32)

    mesh = plsc.VectorSubcoreMesh(core_axis_name="c", subcore_axis_name="s")
    sc_partials = pl.kernel(
        _sc_partials_body,
        out_type=(
            jax.ShapeDtypeStruct((_NW, _C, _LANES), jnp.float32),
            jax.ShapeDtypeStruct((_NW, _C, _LANES), jnp.float32),
            jax.ShapeDtypeStruct((_NW, _LANES), jnp.float32),
        ),
        mesh=mesh,
        scratch_types=[
            pltpu.VMEM((_B * _CHUNK,), jnp.float32),
            pltpu.VMEM((2, _G, _ROWS, _W), jnp.float32),
            pltpu.VMEM((_C, _LANES), jnp.float32),
            pltpu.VMEM((_C, _LANES), jnp.float32),
            pltpu.VMEM((_LANES,), jnp.float32),
            pltpu.SemaphoreType.DMA,
            pltpu.SemaphoreType.DMA,
        ],
    )
    part_s, part_ss, part_n = sc_partials(img, mask_f)

    loss = pl.pallas_call(
        _finalize_body,
        out_shape=jax.ShapeDtypeStruct((1, 1), jnp.float32),
    )(part_s, part_ss, part_n)
    return loss[0, 0]


# speedup vs baseline: 4.0733x; 1.0142x over previous
import jax
import jax.numpy as jnp
from jax import lax
from jax.experimental import pallas as pl
from jax.experimental.pallas import tpu as pltpu
from jax.experimental.pallas import tpu_sc as plsc

_B, _C, _H, _W = 4, 96, 512, 512
_HW = _H * _W
_NC, _NS = 2, 16
_NW = _NC * _NS
_ROWS = _H // _NW
_CHUNK = _ROWS * _W
_LANES = 16
_G = 8
_NG = _C // _G
_HROWS = _ROWS // 2
_HCHUNK = _HROWS * _W
_UNROLL = 4
_STEPS = _HCHUNK // (_LANES * _UNROLL)
_VPR = _W // _LANES


def _sc_partials_body(img, mask, out_s, out_ss, out_n,
                      mask_v, buf, s_v, ss_v, n_v, sem0, sem1):
    wid = lax.axis_index("s") * _NC + lax.axis_index("c")
    off = wid * _CHUNK
    row0 = wid * _ROWS
    sems = (sem0, sem1)
    zero = jnp.zeros((_LANES,), jnp.float32)

    for b in range(_B):
        pltpu.sync_copy(mask.at[b, pl.ds(off, _CHUNK)],
                        mask_v.at[pl.ds(b * _CHUNK, _CHUNK)])

    def issue(b, h, c0, slot):
        pltpu.async_copy(
            img.at[b, pl.ds(c0, _G), pl.ds(row0 + h * _HROWS, _HROWS), :],
            buf.at[slot], sems[slot])

    issue(0, 0, 0, 0)
    issue(0, 1, 0, 1)

    def n_body(i, acc):
        return acc + mask_v[pl.ds(i * _LANES, _LANES)]
    n_v[...] = lax.fori_loop(0, (_B * _CHUNK) // _LANES, n_body, zero)

    def group(gi, carry):
        del carry
        c0 = gi * _G
        acc = [zero] * (2 * _G)
        for b in range(_B):
            for h in range(2):
                slot = h
                pltpu.make_async_copy(
                    img.at[0, pl.ds(0, _G), pl.ds(0, _HROWS), :],
                    buf.at[slot], sems[slot]).wait()
                mbase = b * _CHUNK + h * _HCHUNK

                def step(i, a):
                    a = list(a)
                    r = lax.div(i * _UNROLL, _VPR)
                    cb = lax.rem(i * _UNROLL, _VPR) * _LANES
                    fb = mbase + i * (_LANES * _UNROLL)
                    for u in range(_UNROLL):
                        m = mask_v[pl.ds(fb + u * _LANES, _LANES)]
                        for g in range(_G):
                            x = buf[slot, g, r,
                                    pl.ds(cb + u * _LANES, _LANES)]
                            xm = x * m
                            a[2 * g] = a[2 * g] + xm
                            a[2 * g + 1] = a[2 * g + 1] + xm * xm
                    return tuple(a)

                acc = list(lax.fori_loop(0, _STEPS, step, tuple(acc)))

                if b < _B - 1:
                    issue(b + 1, h, c0, slot)
                else:
                    @pl.when(gi + 1 < _NG)
                    def _():
                        issue(0, h, c0 + _G, slot)

        for g in range(_G):
            s_v[c0 + g] = acc[2 * g]
            ss_v[c0 + g] = acc[2 * g + 1]
        return 0

    lax.fori_loop(0, _NG, group, 0)

    pltpu.sync_copy(s_v, out_s.at[wid])
    pltpu.sync_copy(ss_v, out_ss.at[wid])
    pltpu.sync_copy(n_v, out_n.at[wid])


def _finalize_body(s_ref, ss_ref, n_ref, out_ref):
    s = jnp.sum(s_ref[...], axis=(0, 2))
    ss = jnp.sum(ss_ref[...], axis=(0, 2))
    n = jnp.sum(n_ref[...])
    var = (ss - s * s / n) / (n - 1.0)
    std = jnp.sqrt(var)
    out_ref[...] = (jnp.sum(std) / _C).reshape(1, 1)


@jax.jit
def kernel(img, mask):
    mask_f = mask.reshape(_B, _HW).astype(jnp.float32)

    mesh = plsc.VectorSubcoreMesh(core_axis_name="c", subcore_axis_name="s")
    sc_partials = pl.kernel(
        _sc_partials_body,
        out_type=(
            jax.ShapeDtypeStruct((_NW, _C, _LANES), jnp.float32),
            jax.ShapeDtypeStruct((_NW, _C, _LANES), jnp.float32),
            jax.ShapeDtypeStruct((_NW, _LANES), jnp.float32),
        ),
        mesh=mesh,
        scratch_types=[
            pltpu.VMEM((_B * _CHUNK,), jnp.float32),
            pltpu.VMEM((2, _G, _HROWS, _W), jnp.float32),
            pltpu.VMEM((_C, _LANES), jnp.float32),
            pltpu.VMEM((_C, _LANES), jnp.float32),
            pltpu.VMEM((_LANES,), jnp.float32),
            pltpu.SemaphoreType.DMA,
            pltpu.SemaphoreType.DMA,
        ],
    )
    part_s, part_ss, part_n = sc_partials(img, mask_f)

    loss = pl.pallas_call(
        _finalize_body,
        out_shape=jax.ShapeDtypeStruct((1, 1), jnp.float32),
    )(part_s, part_ss, part_n)
    return loss[0, 0]


# speedup vs baseline: 4.7223x; 1.1593x over previous
import jax
import jax.numpy as jnp
from jax import lax
from jax.experimental import pallas as pl
from jax.experimental.pallas import tpu as pltpu
from jax.experimental.pallas import tpu_sc as plsc

_B, _C, _H, _W = 4, 96, 512, 512
_HW = _H * _W
_C_SC = 32
_C_TC = _C - _C_SC
_NC, _NS = 2, 16
_NW = _NC * _NS
_ROWS = _H // _NW
_CHUNK = _ROWS * _W
_LANES = 16
_G = 8
_NG = _C_SC // _G
_HROWS = _ROWS // 2
_HCHUNK = _HROWS * _W
_UNROLL = 4
_STEPS = _HCHUNK // (_LANES * _UNROLL)
_VPR = _W // _LANES

_CBLK = 32
_HBLK = 32
_NCB = _C_TC // _CBLK
_NHB = _H // _HBLK


def _sc_partials_body(img, mask, out_s, out_ss, out_n,
                      mask_v, buf, s_v, ss_v, n_v, sem0, sem1):
    wid = lax.axis_index("s") * _NC + lax.axis_index("c")
    off = wid * _CHUNK
    row0 = wid * _ROWS
    sems = (sem0, sem1)
    zero = jnp.zeros((_LANES,), jnp.float32)

    for b in range(_B):
        pltpu.sync_copy(mask.at[b, pl.ds(off, _CHUNK)],
                        mask_v.at[pl.ds(b * _CHUNK, _CHUNK)])

    def issue(b, h, c0, slot):
        pltpu.async_copy(
            img.at[b, pl.ds(c0, _G), pl.ds(row0 + h * _HROWS, _HROWS), :],
            buf.at[slot], sems[slot])

    issue(0, 0, 0, 0)
    issue(0, 1, 0, 1)

    def n_body(i, acc):
        return acc + mask_v[pl.ds(i * _LANES, _LANES)]
    n_v[...] = lax.fori_loop(0, (_B * _CHUNK) // _LANES, n_body, zero)

    def group(gi, carry):
        del carry
        c0 = gi * _G
        acc = [zero] * (2 * _G)
        for b in range(_B):
            for h in range(2):
                slot = h
                pltpu.make_async_copy(
                    img.at[0, pl.ds(0, _G), pl.ds(0, _HROWS), :],
                    buf.at[slot], sems[slot]).wait()
                mbase = b * _CHUNK + h * _HCHUNK

                def step(i, a):
                    a = list(a)
                    r = lax.div(i * _UNROLL, _VPR)
                    cb = lax.rem(i * _UNROLL, _VPR) * _LANES
                    fb = mbase + i * (_LANES * _UNROLL)
                    for u in range(_UNROLL):
                        m = mask_v[pl.ds(fb + u * _LANES, _LANES)]
                        for g in range(_G):
                            x = buf[slot, g, r,
                                    pl.ds(cb + u * _LANES, _LANES)]
                            xm = x * m
                            a[2 * g] = a[2 * g] + xm
                            a[2 * g + 1] = a[2 * g + 1] + xm * xm
                    return tuple(a)

                acc = list(lax.fori_loop(0, _STEPS, step, tuple(acc)))

                if b < _B - 1:
                    issue(b + 1, h, c0, slot)
                else:
                    @pl.when(gi + 1 < _NG)
                    def _():
                        issue(0, h, c0 + _G, slot)

        for g in range(_G):
            s_v[c0 + g] = acc[2 * g]
            ss_v[c0 + g] = acc[2 * g + 1]
        return 0

    lax.fori_loop(0, _NG, group, 0)

    pltpu.sync_copy(s_v, out_s.at[wid])
    pltpu.sync_copy(ss_v, out_ss.at[wid])
    pltpu.sync_copy(n_v, out_n.at[wid])


def _tc_partials_body(img_ref, mask_ref, out_ref):
    b = pl.program_id(0)
    cb = pl.program_id(1)
    hi = pl.program_id(2)

    @pl.when((b == 0) & (cb == 0) & (hi == 0))
    def _():
        out_ref[...] = jnp.zeros_like(out_ref)

    x = img_ref[0]
    m = mask_ref[0]
    xm = x * m
    s_p = jnp.sum(xm, axis=(1, 2))
    ss_p = jnp.sum(xm * xm, axis=(1, 2))
    out_ref[0, cb] += s_p
    out_ref[1, cb] += ss_p


def _finalize_body(s_ref, ss_ref, n_ref, tc_ref, out_ref):
    n = jnp.sum(n_ref[...])
    s_sc = jnp.sum(s_ref[...], axis=(0, 2))
    ss_sc = jnp.sum(ss_ref[...], axis=(0, 2))
    var_sc = (ss_sc - s_sc * s_sc / n) / (n - 1.0)
    s_tc = tc_ref[0]
    ss_tc = tc_ref[1]
    var_tc = (ss_tc - s_tc * s_tc / n) / (n - 1.0)
    total = jnp.sum(jnp.sqrt(var_sc)) + jnp.sum(jnp.sqrt(var_tc))
    out_ref[...] = (total / _C).reshape(1, 1)


@jax.jit
def kernel(img, mask):
    mask_f = mask.reshape(_B, _HW).astype(jnp.float32)
    mask_f4 = mask.astype(jnp.float32)

    mesh = plsc.VectorSubcoreMesh(core_axis_name="c", subcore_axis_name="s")
    sc_partials = pl.kernel(
        _sc_partials_body,
        out_type=(
            jax.ShapeDtypeStruct((_NW, _C_SC, _LANES), jnp.float32),
            jax.ShapeDtypeStruct((_NW, _C_SC, _LANES), jnp.float32),
            jax.ShapeDtypeStruct((_NW, _LANES), jnp.float32),
        ),
        mesh=mesh,
        scratch_types=[
            pltpu.VMEM((_B * _CHUNK,), jnp.float32),
            pltpu.VMEM((2, _G, _HROWS, _W), jnp.float32),
            pltpu.VMEM((_C_SC, _LANES), jnp.float32),
            pltpu.VMEM((_C_SC, _LANES), jnp.float32),
            pltpu.VMEM((_LANES,), jnp.float32),
            pltpu.SemaphoreType.DMA,
            pltpu.SemaphoreType.DMA,
        ],
    )
    part_s, part_ss, part_n = sc_partials(img, mask_f)

    tc_part = pl.pallas_call(
        _tc_partials_body,
        grid=(_B, _NCB, _NHB),
        in_specs=[
            pl.BlockSpec((1, _CBLK, _HBLK, _W),
                         lambda b, cb, hi: (b, cb + _C_SC // _CBLK, hi, 0)),
            pl.BlockSpec((1, 1, _HBLK, _W),
                         lambda b, cb, hi: (b, 0, hi, 0)),
        ],
        out_specs=pl.BlockSpec((2, _NCB, _CBLK), lambda b, cb, hi: (0, 0, 0)),
        out_shape=jax.ShapeDtypeStruct((2, _NCB, _CBLK), jnp.float32),
        compiler_params=pltpu.CompilerParams(
            dimension_semantics=("arbitrary", "arbitrary", "arbitrary"),
        ),
    )(img, mask_f4)

    loss = pl.pallas_call(
        _finalize_body,
        out_shape=jax.ShapeDtypeStruct((1, 1), jnp.float32),
    )(part_s, part_ss, part_n, tc_part)
    return loss[0, 0]


# speedup vs baseline: 4.7455x; 1.0049x over previous
import jax
import jax.numpy as jnp
from jax import lax
from jax.experimental import pallas as pl
from jax.experimental.pallas import tpu as pltpu
from jax.experimental.pallas import tpu_sc as plsc

_B, _C, _H, _W = 4, 96, 512, 512
_HW = _H * _W
_C_SC = 32
_C_TC = _C - _C_SC
_NC, _NS = 2, 16
_NW = _NC * _NS
_ROWS = _H // _NW
_CHUNK = _ROWS * _W
_LANES = 16
_G = 8
_NG = _C_SC // _G
_HROWS = _ROWS // 2
_HCHUNK = _HROWS * _W
_UNROLL = 4
_STEPS = _HCHUNK // (_LANES * _UNROLL)
_VPR = _W // _LANES

_CBLK = 32
_HBLK = 32
_NCB = _C_TC // _CBLK
_NHB = _H // _HBLK


def _sc_partials_body(img, mask, out_s, out_ss, out_n,
                      mask_v, buf, s_v, ss_v, n_v, sem0, sem1):
    wid = lax.axis_index("s") * _NC + lax.axis_index("c")
    off = wid * _CHUNK
    row0 = wid * _ROWS
    sems = (sem0, sem1)
    zero = jnp.zeros((_LANES,), jnp.float32)

    for b in range(_B):
        pltpu.sync_copy(mask.at[b, pl.ds(off, _CHUNK)],
                        mask_v.at[pl.ds(b * _CHUNK, _CHUNK)])

    def issue(b, h, c0, slot):
        pltpu.async_copy(
            img.at[b, pl.ds(c0, _G), pl.ds(row0 + h * _HROWS, _HROWS), :],
            buf.at[slot], sems[slot])

    issue(0, 0, 0, 0)
    issue(0, 1, 0, 1)

    def n_body(i, acc):
        return acc + mask_v[pl.ds(i * _LANES, _LANES)]
    n_v[...] = lax.fori_loop(0, (_B * _CHUNK) // _LANES, n_body, zero)

    def group(gi, carry):
        del carry
        c0 = gi * _G
        acc = [zero] * (2 * _G)
        for b in range(_B):
            for h in range(2):
                slot = h
                pltpu.make_async_copy(
                    img.at[0, pl.ds(0, _G), pl.ds(0, _HROWS), :],
                    buf.at[slot], sems[slot]).wait()
                mbase = b * _CHUNK + h * _HCHUNK

                def step(i, a):
                    a = list(a)
                    r = lax.div(i * _UNROLL, _VPR)
                    cb = lax.rem(i * _UNROLL, _VPR) * _LANES
                    fb = mbase + i * (_LANES * _UNROLL)
                    for u in range(_UNROLL):
                        m = mask_v[pl.ds(fb + u * _LANES, _LANES)]
                        for g in range(_G):
                            x = buf[slot, g, r,
                                    pl.ds(cb + u * _LANES, _LANES)]
                            xm = x * m
                            a[2 * g] = a[2 * g] + xm
                            a[2 * g + 1] = a[2 * g + 1] + xm * xm
                    return tuple(a)

                acc = list(lax.fori_loop(0, _STEPS, step, tuple(acc)))

                if b < _B - 1:
                    issue(b + 1, h, c0, slot)
                else:
                    @pl.when(gi + 1 < _NG)
                    def _():
                        issue(0, h, c0 + _G, slot)

        for g in range(_G):
            s_v[c0 + g] = acc[2 * g]
            ss_v[c0 + g] = acc[2 * g + 1]
        return 0

    lax.fori_loop(0, _NG, group, 0)

    pltpu.sync_copy(s_v, out_s.at[wid])
    pltpu.sync_copy(ss_v, out_ss.at[wid])
    pltpu.sync_copy(n_v, out_n.at[wid])


def _tc_partials_body(img_ref, mask_ref, out_ref):
    b = pl.program_id(0)
    cb = pl.program_id(1)
    hi = pl.program_id(2)

    @pl.when((b == 0) & (cb == 0) & (hi == 0))
    def _():
        out_ref[...] = jnp.zeros_like(out_ref)

    x = img_ref[0]
    m = mask_ref[0]
    xm = x * m
    s4 = jnp.sum(xm.reshape(_CBLK, _HBLK // 8, 8, _W), axis=1)
    ss4 = jnp.sum((xm * xm).reshape(_CBLK, _HBLK // 8, 8, _W), axis=1)
    out_ref[0, cb] += s4
    out_ref[1, cb] += ss4


def _finalize_body(s_ref, ss_ref, n_ref, tc_ref, out_ref):
    n = jnp.sum(n_ref[...])
    s_sc = jnp.sum(s_ref[...], axis=(0, 2))
    ss_sc = jnp.sum(ss_ref[...], axis=(0, 2))
    var_sc = (ss_sc - s_sc * s_sc / n) / (n - 1.0)
    s_tc = jnp.sum(jnp.sum(tc_ref[0], axis=3), axis=2)
    ss_tc = jnp.sum(jnp.sum(tc_ref[1], axis=3), axis=2)
    var_tc = (ss_tc - s_tc * s_tc / n) / (n - 1.0)
    total = jnp.sum(jnp.sqrt(var_sc)) + jnp.sum(jnp.sqrt(var_tc))
    out_ref[...] = (total / _C).reshape(1, 1)


@jax.jit
def kernel(img, mask):
    mask_f = mask.reshape(_B, _HW).astype(jnp.float32)
    mask_f4 = mask.astype(jnp.float32)

    mesh = plsc.VectorSubcoreMesh(core_axis_name="c", subcore_axis_name="s")
    sc_partials = pl.kernel(
        _sc_partials_body,
        out_type=(
            jax.ShapeDtypeStruct((_NW, _C_SC, _LANES), jnp.float32),
            jax.ShapeDtypeStruct((_NW, _C_SC, _LANES), jnp.float32),
            jax.ShapeDtypeStruct((_NW, _LANES), jnp.float32),
        ),
        mesh=mesh,
        scratch_types=[
            pltpu.VMEM((_B * _CHUNK,), jnp.float32),
            pltpu.VMEM((2, _G, _HROWS, _W), jnp.float32),
            pltpu.VMEM((_C_SC, _LANES), jnp.float32),
            pltpu.VMEM((_C_SC, _LANES), jnp.float32),
            pltpu.VMEM((_LANES,), jnp.float32),
            pltpu.SemaphoreType.DMA,
            pltpu.SemaphoreType.DMA,
        ],
    )
    part_s, part_ss, part_n = sc_partials(img, mask_f)

    tc_part = pl.pallas_call(
        _tc_partials_body,
        grid=(_B, _NCB, _NHB),
        in_specs=[
            pl.BlockSpec((1, _CBLK, _HBLK, _W),
                         lambda b, cb, hi: (b, cb + _C_SC // _CBLK, hi, 0)),
            pl.BlockSpec((1, 1, _HBLK, _W),
                         lambda b, cb, hi: (b, 0, hi, 0)),
        ],
        out_specs=pl.BlockSpec((2, _NCB, _CBLK, 8, _W),
                               lambda b, cb, hi: (0, 0, 0, 0, 0)),
        out_shape=jax.ShapeDtypeStruct((2, _NCB, _CBLK, 8, _W), jnp.float32),
        compiler_params=pltpu.CompilerParams(
            dimension_semantics=("arbitrary", "arbitrary", "arbitrary"),
        ),
    )(img, mask_f4)

    loss = pl.pallas_call(
        _finalize_body,
        out_shape=jax.ShapeDtypeStruct((1, 1), jnp.float32),
    )(part_s, part_ss, part_n, tc_part)
    return loss[0, 0]


# speedup vs baseline: 5.6663x; 1.1940x over previous
import jax
import jax.numpy as jnp
from jax import lax
from jax.experimental import pallas as pl
from jax.experimental.pallas import tpu as pltpu
from jax.experimental.pallas import tpu_sc as plsc

_B, _C, _H, _W = 4, 96, 512, 512
_HW = _H * _W
_C_SC = 32
_C_TC = _C - _C_SC
_NC, _NS = 2, 16
_NW = _NC * _NS
_ROWS = _H // _NW
_CHUNK = _ROWS * _W
_LANES = 16
_G = 8
_NG = _C_SC // _G
_HROWS = _ROWS // 2
_HCHUNK = _HROWS * _W
_UNROLL = 4
_STEPS = _HCHUNK // (_LANES * _UNROLL)
_VPR = _W // _LANES

_CGRP = 8
_NCG = _C_TC // _CGRP


def _sc_partials_body(img, mask, out_s, out_ss, out_n,
                      mask_v, buf, s_v, ss_v, n_v, sem0, sem1):
    wid = lax.axis_index("s") * _NC + lax.axis_index("c")
    off = wid * _CHUNK
    row0 = wid * _ROWS
    sems = (sem0, sem1)
    zero = jnp.zeros((_LANES,), jnp.float32)

    for b in range(_B):
        pltpu.sync_copy(mask.at[b, pl.ds(off, _CHUNK)],
                        mask_v.at[pl.ds(b * _CHUNK, _CHUNK)])

    def issue(b, h, c0, slot):
        pltpu.async_copy(
            img.at[b, pl.ds(c0, _G), pl.ds(row0 + h * _HROWS, _HROWS), :],
            buf.at[slot], sems[slot])

    issue(0, 0, 0, 0)
    issue(0, 1, 0, 1)

    def n_body(i, acc):
        return acc + mask_v[pl.ds(i * _LANES, _LANES)]
    n_v[...] = lax.fori_loop(0, (_B * _CHUNK) // _LANES, n_body, zero)

    def group(gi, carry):
        del carry
        c0 = gi * _G
        acc = [zero] * (2 * _G)
        for b in range(_B):
            for h in range(2):
                slot = h
                pltpu.make_async_copy(
                    img.at[0, pl.ds(0, _G), pl.ds(0, _HROWS), :],
                    buf.at[slot], sems[slot]).wait()
                mbase = b * _CHUNK + h * _HCHUNK

                def step(i, a):
                    a = list(a)
                    r = lax.div(i * _UNROLL, _VPR)
                    cb = lax.rem(i * _UNROLL, _VPR) * _LANES
                    fb = mbase + i * (_LANES * _UNROLL)
                    for u in range(_UNROLL):
                        m = mask_v[pl.ds(fb + u * _LANES, _LANES)]
                        for g in range(_G):
                            x = buf[slot, g, r,
                                    pl.ds(cb + u * _LANES, _LANES)]
                            xm = x * m
                            a[2 * g] = a[2 * g] + xm
                            a[2 * g + 1] = a[2 * g + 1] + xm * xm
                    return tuple(a)

                acc = list(lax.fori_loop(0, _STEPS, step, tuple(acc)))

                if b < _B - 1:
                    issue(b + 1, h, c0, slot)
                else:
                    @pl.when(gi + 1 < _NG)
                    def _():
                        issue(0, h, c0 + _G, slot)

        for g in range(_G):
            s_v[c0 + g] = acc[2 * g]
            ss_v[c0 + g] = acc[2 * g + 1]
        return 0

    lax.fori_loop(0, _NG, group, 0)

    pltpu.sync_copy(s_v, out_s.at[wid])
    pltpu.sync_copy(ss_v, out_ss.at[wid])
    pltpu.sync_copy(n_v, out_n.at[wid])


def _tc_partials_body(img_ref, mask_ref, out_ref):
    b = pl.program_id(0)
    cg = pl.program_id(1)

    @pl.when((b == 0) & (cg == 0))
    def _():
        out_ref[...] = jnp.zeros_like(out_ref)

    x = img_ref[0]
    m = mask_ref[0]
    xm = x * m
    s4 = jnp.sum(xm.reshape(_CGRP, _H // 8, 8, _W), axis=1)
    ss4 = jnp.sum((xm * xm).reshape(_CGRP, _H // 8, 8, _W), axis=1)
    out_ref[0, cg] += s4
    out_ref[1, cg] += ss4


def _finalize_body(s_ref, ss_ref, n_ref, tc_ref, out_ref):
    n = jnp.sum(n_ref[...])
    s_sc = jnp.sum(s_ref[...], axis=(0, 2))
    ss_sc = jnp.sum(ss_ref[...], axis=(0, 2))
    var_sc = (ss_sc - s_sc * s_sc / n) / (n - 1.0)
    s_tc = jnp.sum(jnp.sum(tc_ref[0], axis=3), axis=2)
    ss_tc = jnp.sum(jnp.sum(tc_ref[1], axis=3), axis=2)
    var_tc = (ss_tc - s_tc * s_tc / n) / (n - 1.0)
    total = jnp.sum(jnp.sqrt(var_sc)) + jnp.sum(jnp.sqrt(var_tc))
    out_ref[...] = (total / _C).reshape(1, 1)


@jax.jit
def kernel(img, mask):
    mask_f = mask.reshape(_B, _HW).astype(jnp.float32)
    mask_f4 = mask.astype(jnp.float32)

    mesh = plsc.VectorSubcoreMesh(core_axis_name="c", subcore_axis_name="s")
    sc_partials = pl.kernel(
        _sc_partials_body,
        out_type=(
            jax.ShapeDtypeStruct((_NW, _C_SC, _LANES), jnp.float32),
            jax.ShapeDtypeStruct((_NW, _C_SC, _LANES), jnp.float32),
            jax.ShapeDtypeStruct((_NW, _LANES), jnp.float32),
        ),
        mesh=mesh,
        scratch_types=[
            pltpu.VMEM((_B * _CHUNK,), jnp.float32),
            pltpu.VMEM((2, _G, _HROWS, _W), jnp.float32),
            pltpu.VMEM((_C_SC, _LANES), jnp.float32),
            pltpu.VMEM((_C_SC, _LANES), jnp.float32),
            pltpu.VMEM((_LANES,), jnp.float32),
            pltpu.SemaphoreType.DMA,
            pltpu.SemaphoreType.DMA,
        ],
    )
    part_s, part_ss, part_n = sc_partials(img, mask_f)

    tc_part = pl.pallas_call(
        _tc_partials_body,
        grid=(_B, _NCG),
        in_specs=[
            pl.BlockSpec((1, _CGRP, _H, _W),
                         lambda b, cg: (b, cg + _C_SC // _CGRP, 0, 0)),
            pl.BlockSpec((1, 1, _H, _W),
                         lambda b, cg: (b, 0, 0, 0)),
        ],
        out_specs=pl.BlockSpec((2, _NCG, _CGRP, 8, _W),
                               lambda b, cg: (0, 0, 0, 0, 0)),
        out_shape=jax.ShapeDtypeStruct((2, _NCG, _CGRP, 8, _W), jnp.float32),
        compiler_params=pltpu.CompilerParams(
            dimension_semantics=("arbitrary", "arbitrary"),
        ),
    )(img, mask_f4)

    loss = pl.pallas_call(
        _finalize_body,
        out_shape=jax.ShapeDtypeStruct((1, 1), jnp.float32),
    )(part_s, part_ss, part_n, tc_part)
    return loss[0, 0]


# speedup vs baseline: 5.8826x; 1.0382x over previous
import jax
import jax.numpy as jnp
from jax import lax
from jax.experimental import pallas as pl
from jax.experimental.pallas import tpu as pltpu
from jax.experimental.pallas import tpu_sc as plsc

_B, _C, _H, _W = 4, 96, 512, 512
_HW = _H * _W
_C_SC = 32
_C_TC = _C - _C_SC
_NC, _NS = 2, 16
_NW = _NC * _NS
_ROWS = _H // _NW
_CHUNK = _ROWS * _W
_LANES = 16
_G = 8
_NG = _C_SC // _G
_HROWS = _ROWS // 2
_HCHUNK = _HROWS * _W
_UNROLL = 4
_STEPS = _HCHUNK // (_LANES * _UNROLL)
_VPR = _W // _LANES

_CGRP = 8
_NCG = _C_TC // _CGRP


def _sc_partials_body(img, mask, out_s, out_ss, out_n,
                      mask_v, buf, s_v, ss_v, n_v, sem0, sem1):
    wid = lax.axis_index("s") * _NC + lax.axis_index("c")
    off = wid * _CHUNK
    row0 = wid * _ROWS
    sems = (sem0, sem1)
    zero = jnp.zeros((_LANES,), jnp.float32)

    for b in range(_B):
        pltpu.sync_copy(mask.at[b, 0, pl.ds(row0, _ROWS), :],
                        mask_v.at[b])

    def issue(b, h, c0, slot):
        pltpu.async_copy(
            img.at[b, pl.ds(c0, _G), pl.ds(row0 + h * _HROWS, _HROWS), :],
            buf.at[slot], sems[slot])

    issue(0, 0, 0, 0)
    issue(0, 1, 0, 1)

    def n_body(i, acc):
        rr = lax.div(i, _VPR)
        cc = lax.rem(i, _VPR) * _LANES
        acc0, acc1, acc2, acc3 = acc
        return (acc0 + mask_v[0, rr, pl.ds(cc, _LANES)],
                acc1 + mask_v[1, rr, pl.ds(cc, _LANES)],
                acc2 + mask_v[2, rr, pl.ds(cc, _LANES)],
                acc3 + mask_v[3, rr, pl.ds(cc, _LANES)])
    nacc = lax.fori_loop(0, _CHUNK // _LANES, n_body, (zero,) * _B)
    n_v[...] = nacc[0] + nacc[1] + nacc[2] + nacc[3]

    def group(gi, carry):
        del carry
        c0 = gi * _G
        acc = [zero] * (2 * _G)
        for b in range(_B):
            for h in range(2):
                slot = h
                pltpu.make_async_copy(
                    img.at[0, pl.ds(0, _G), pl.ds(0, _HROWS), :],
                    buf.at[slot], sems[slot]).wait()
                def step(i, a):
                    a = list(a)
                    r = lax.div(i * _UNROLL, _VPR)
                    cb = lax.rem(i * _UNROLL, _VPR) * _LANES
                    for u in range(_UNROLL):
                        m = mask_v[b, h * _HROWS + r,
                                   pl.ds(cb + u * _LANES, _LANES)]
                        for g in range(_G):
                            x = buf[slot, g, r,
                                    pl.ds(cb + u * _LANES, _LANES)]
                            xm = x * m
                            a[2 * g] = a[2 * g] + xm
                            a[2 * g + 1] = a[2 * g + 1] + xm * xm
                    return tuple(a)

                acc = list(lax.fori_loop(0, _STEPS, step, tuple(acc)))

                if b < _B - 1:
                    issue(b + 1, h, c0, slot)
                else:
                    @pl.when(gi + 1 < _NG)
                    def _():
                        issue(0, h, c0 + _G, slot)

        for g in range(_G):
            s_v[c0 + g] = acc[2 * g]
            ss_v[c0 + g] = acc[2 * g + 1]
        return 0

    lax.fori_loop(0, _NG, group, 0)

    pltpu.sync_copy(s_v, out_s.at[wid])
    pltpu.sync_copy(ss_v, out_ss.at[wid])
    pltpu.sync_copy(n_v, out_n.at[wid])


def _tc_partials_body(img_ref, mask_ref, out_ref):
    b = pl.program_id(0)
    cg = pl.program_id(1)

    @pl.when((b == 0) & (cg == 0))
    def _():
        out_ref[...] = jnp.zeros_like(out_ref)

    x = img_ref[0]
    m = mask_ref[0]
    xm = x * m
    s4 = jnp.sum(xm.reshape(_CGRP, _H // 8, 8, _W), axis=1)
    ss4 = jnp.sum((xm * xm).reshape(_CGRP, _H // 8, 8, _W), axis=1)
    s2 = jnp.sum(s4.reshape(_CGRP, 8, _W // 128, 128), axis=2)
    ss2 = jnp.sum(ss4.reshape(_CGRP, 8, _W // 128, 128), axis=2)
    out_ref[0, cg] += s2
    out_ref[1, cg] += ss2


def _finalize_body(s_ref, ss_ref, n_ref, tc_ref, out_ref):
    n = jnp.sum(n_ref[...])
    s_sc = jnp.sum(s_ref[...], axis=(0, 2))
    ss_sc = jnp.sum(ss_ref[...], axis=(0, 2))
    var_sc = (ss_sc - s_sc * s_sc / n) / (n - 1.0)
    s_tc = jnp.sum(jnp.sum(tc_ref[0], axis=3), axis=2)
    ss_tc = jnp.sum(jnp.sum(tc_ref[1], axis=3), axis=2)
    var_tc = (ss_tc - s_tc * s_tc / n) / (n - 1.0)
    total = jnp.sum(jnp.sqrt(var_sc)) + jnp.sum(jnp.sqrt(var_tc))
    out_ref[...] = (total / _C).reshape(1, 1)


@jax.jit
def kernel(img, mask):
    mask_f4 = mask.astype(jnp.float32)

    mesh = plsc.VectorSubcoreMesh(core_axis_name="c", subcore_axis_name="s")
    sc_partials = pl.kernel(
        _sc_partials_body,
        out_type=(
            jax.ShapeDtypeStruct((_NW, _C_SC, _LANES), jnp.float32),
            jax.ShapeDtypeStruct((_NW, _C_SC, _LANES), jnp.float32),
            jax.ShapeDtypeStruct((_NW, _LANES), jnp.float32),
        ),
        mesh=mesh,
        scratch_types=[
            pltpu.VMEM((_B, _ROWS, _W), jnp.float32),
            pltpu.VMEM((2, _G, _HROWS, _W), jnp.float32),
            pltpu.VMEM((_C_SC, _LANES), jnp.float32),
            pltpu.VMEM((_C_SC, _LANES), jnp.float32),
            pltpu.VMEM((_LANES,), jnp.float32),
            pltpu.SemaphoreType.DMA,
            pltpu.SemaphoreType.DMA,
        ],
    )
    part_s, part_ss, part_n = sc_partials(img, mask_f4)

    tc_part = pl.pallas_call(
        _tc_partials_body,
        grid=(_B, _NCG),
        in_specs=[
            pl.BlockSpec((1, _CGRP, _H, _W),
                         lambda b, cg: (b, cg + _C_SC // _CGRP, 0, 0)),
            pl.BlockSpec((1, 1, _H, _W),
                         lambda b, cg: (b, 0, 0, 0)),
        ],
        out_specs=pl.BlockSpec((2, _NCG, _CGRP, 8, 128),
                               lambda b, cg: (0, 0, 0, 0, 0)),
        out_shape=jax.ShapeDtypeStruct((2, _NCG, _CGRP, 8, 128), jnp.float32),
        compiler_params=pltpu.CompilerParams(
            dimension_semantics=("arbitrary", "arbitrary"),
        ),
    )(img, mask_f4)

    loss = pl.pallas_call(
        _finalize_body,
        out_shape=jax.ShapeDtypeStruct((1, 1), jnp.float32),
    )(part_s, part_ss, part_n, tc_part)
    return loss[0, 0]


# speedup vs baseline: 5.8930x; 1.0018x over previous
import jax
import jax.numpy as jnp
from jax import lax
from jax.experimental import pallas as pl
from jax.experimental.pallas import tpu as pltpu
from jax.experimental.pallas import tpu_sc as plsc

_B, _C, _H, _W = 4, 96, 512, 512
_HW = _H * _W
_C_SC = 40
_C_TC = _C - _C_SC
_NC, _NS = 2, 16
_NW = _NC * _NS
_ROWS = _H // _NW
_CHUNK = _ROWS * _W
_LANES = 16
_G = 8
_NG = _C_SC // _G
_HROWS = _ROWS // 2
_HCHUNK = _HROWS * _W
_UNROLL = 4
_STEPS = _HCHUNK // (_LANES * _UNROLL)
_VPR = _W // _LANES

_CGRP = 8
_NCG = _C_TC // _CGRP


def _sc_partials_body(img, mask, out_s, out_ss, out_n,
                      mask_v, buf, s_v, ss_v, n_v, sem0, sem1):
    wid = lax.axis_index("s") * _NC + lax.axis_index("c")
    off = wid * _CHUNK
    row0 = wid * _ROWS
    sems = (sem0, sem1)
    zero = jnp.zeros((_LANES,), jnp.float32)

    for b in range(_B):
        pltpu.sync_copy(mask.at[b, 0, pl.ds(row0, _ROWS), :],
                        mask_v.at[b])

    def issue(b, h, c0, slot):
        pltpu.async_copy(
            img.at[b, pl.ds(c0, _G), pl.ds(row0 + h * _HROWS, _HROWS), :],
            buf.at[slot], sems[slot])

    issue(0, 0, 0, 0)
    issue(0, 1, 0, 1)

    def n_body(i, acc):
        rr = lax.div(i, _VPR)
        cc = lax.rem(i, _VPR) * _LANES
        acc0, acc1, acc2, acc3 = acc
        return (acc0 + mask_v[0, rr, pl.ds(cc, _LANES)],
                acc1 + mask_v[1, rr, pl.ds(cc, _LANES)],
                acc2 + mask_v[2, rr, pl.ds(cc, _LANES)],
                acc3 + mask_v[3, rr, pl.ds(cc, _LANES)])
    nacc = lax.fori_loop(0, _CHUNK // _LANES, n_body, (zero,) * _B)
    n_v[...] = nacc[0] + nacc[1] + nacc[2] + nacc[3]

    def group(gi, carry):
        del carry
        c0 = gi * _G
        acc = [zero] * (2 * _G)
        for b in range(_B):
            for h in range(2):
                slot = h
                pltpu.make_async_copy(
                    img.at[0, pl.ds(0, _G), pl.ds(0, _HROWS), :],
                    buf.at[slot], sems[slot]).wait()
                def step(i, a):
                    a = list(a)
                    r = lax.div(i * _UNROLL, _VPR)
                    cb = lax.rem(i * _UNROLL, _VPR) * _LANES
                    for u in range(_UNROLL):
                        m = mask_v[b, h * _HROWS + r,
                                   pl.ds(cb + u * _LANES, _LANES)]
                        for g in range(_G):
                            x = buf[slot, g, r,
                                    pl.ds(cb + u * _LANES, _LANES)]
                            xm = x * m
                            a[2 * g] = a[2 * g] + xm
                            a[2 * g + 1] = a[2 * g + 1] + xm * xm
                    return tuple(a)

                acc = list(lax.fori_loop(0, _STEPS, step, tuple(acc)))

                if b < _B - 1:
                    issue(b + 1, h, c0, slot)
                else:
                    @pl.when(gi + 1 < _NG)
                    def _():
                        issue(0, h, c0 + _G, slot)

        for g in range(_G):
            s_v[c0 + g] = acc[2 * g]
            ss_v[c0 + g] = acc[2 * g + 1]
        return 0

    lax.fori_loop(0, _NG, group, 0)

    pltpu.sync_copy(s_v, out_s.at[wid])
    pltpu.sync_copy(ss_v, out_ss.at[wid])
    pltpu.sync_copy(n_v, out_n.at[wid])


def _tc_partials_body(img_ref, mask_ref, out_ref):
    b = pl.program_id(0)
    cg = pl.program_id(1)

    @pl.when((b == 0) & (cg == 0))
    def _():
        out_ref[...] = jnp.zeros_like(out_ref)

    x = img_ref[0]
    m = mask_ref[0]
    xm = x * m
    s4 = jnp.sum(xm.reshape(_CGRP, _H // 8, 8, _W), axis=1)
    ss4 = jnp.sum((xm * xm).reshape(_CGRP, _H // 8, 8, _W), axis=1)
    s2 = jnp.sum(s4.reshape(_CGRP, 8, _W // 128, 128), axis=2)
    ss2 = jnp.sum(ss4.reshape(_CGRP, 8, _W // 128, 128), axis=2)
    out_ref[0, cg] += s2
    out_ref[1, cg] += ss2


def _finalize_body(s_ref, ss_ref, n_ref, tc_ref, out_ref):
    n = jnp.sum(n_ref[...])
    s_sc = jnp.sum(s_ref[...], axis=(0, 2))
    ss_sc = jnp.sum(ss_ref[...], axis=(0, 2))
    var_sc = (ss_sc - s_sc * s_sc / n) / (n - 1.0)
    s_tc = jnp.sum(jnp.sum(tc_ref[0], axis=3), axis=2)
    ss_tc = jnp.sum(jnp.sum(tc_ref[1], axis=3), axis=2)
    var_tc = (ss_tc - s_tc * s_tc / n) / (n - 1.0)
    total = jnp.sum(jnp.sqrt(var_sc)) + jnp.sum(jnp.sqrt(var_tc))
    out_ref[...] = (total / _C).reshape(1, 1)


@jax.jit
def kernel(img, mask):
    mask_f4 = mask.astype(jnp.float32)

    mesh = plsc.VectorSubcoreMesh(core_axis_name="c", subcore_axis_name="s")
    sc_partials = pl.kernel(
        _sc_partials_body,
        out_type=(
            jax.ShapeDtypeStruct((_NW, _C_SC, _LANES), jnp.float32),
            jax.ShapeDtypeStruct((_NW, _C_SC, _LANES), jnp.float32),
            jax.ShapeDtypeStruct((_NW, _LANES), jnp.float32),
        ),
        mesh=mesh,
        scratch_types=[
            pltpu.VMEM((_B, _ROWS, _W), jnp.float32),
            pltpu.VMEM((2, _G, _HROWS, _W), jnp.float32),
            pltpu.VMEM((_C_SC, _LANES), jnp.float32),
            pltpu.VMEM((_C_SC, _LANES), jnp.float32),
            pltpu.VMEM((_LANES,), jnp.float32),
            pltpu.SemaphoreType.DMA,
            pltpu.SemaphoreType.DMA,
        ],
    )
    part_s, part_ss, part_n = sc_partials(img, mask_f4)

    tc_part = pl.pallas_call(
        _tc_partials_body,
        grid=(_B, _NCG),
        in_specs=[
            pl.BlockSpec((1, _CGRP, _H, _W),
                         lambda b, cg: (b, cg + _C_SC // _CGRP, 0, 0)),
            pl.BlockSpec((1, 1, _H, _W),
                         lambda b, cg: (b, 0, 0, 0)),
        ],
        out_specs=pl.BlockSpec((2, _NCG, _CGRP, 8, 128),
                               lambda b, cg: (0, 0, 0, 0, 0)),
        out_shape=jax.ShapeDtypeStruct((2, _NCG, _CGRP, 8, 128), jnp.float32),
        compiler_params=pltpu.CompilerParams(
            dimension_semantics=("arbitrary", "arbitrary"),
        ),
    )(img, mask_f4)

    loss = pl.pallas_call(
        _finalize_body,
        out_shape=jax.ShapeDtypeStruct((1, 1), jnp.float32),
    )(part_s, part_ss, part_n, tc_part)
    return loss[0, 0]


# speedup vs baseline: 5.8960x; 1.0005x over previous
import jax
import jax.numpy as jnp
from jax import lax
from jax.experimental import pallas as pl
from jax.experimental.pallas import tpu as pltpu
from jax.experimental.pallas import tpu_sc as plsc

_B, _C, _H, _W = 4, 96, 512, 512
_HW = _H * _W
_C_SC = 40
_C_TC = _C - _C_SC
_NC, _NS = 2, 16
_NW = _NC * _NS
_ROWS = _H // _NW
_CHUNK = _ROWS * _W
_LANES = 16
_G = 4
_NG = _C_SC // _G
_HROWS = _ROWS // 2
_HCHUNK = _HROWS * _W
_UNROLL = 2
_STEPS = _HCHUNK // (_LANES * _UNROLL)
_VPR = _W // _LANES

_CGRP = 8
_NCG = _C_TC // _CGRP


def _sc_partials_body(img, mask, out_s, out_ss, out_n,
                      mask_v, buf, s_v, ss_v, n_v, sem0, sem1):
    wid = lax.axis_index("s") * _NC + lax.axis_index("c")
    off = wid * _CHUNK
    row0 = wid * _ROWS
    sems = (sem0, sem1)
    zero = jnp.zeros((_LANES,), jnp.float32)

    for b in range(_B):
        pltpu.sync_copy(mask.at[b, 0, pl.ds(row0, _ROWS), :],
                        mask_v.at[b])

    def issue(b, h, c0, slot):
        pltpu.async_copy(
            img.at[b, pl.ds(c0, _G), pl.ds(row0 + h * _HROWS, _HROWS), :],
            buf.at[slot], sems[slot])

    issue(0, 0, 0, 0)
    issue(0, 1, 0, 1)

    def n_body(i, acc):
        rr = lax.div(i, _VPR)
        cc = lax.rem(i, _VPR) * _LANES
        acc0, acc1, acc2, acc3 = acc
        return (acc0 + mask_v[0, rr, pl.ds(cc, _LANES)],
                acc1 + mask_v[1, rr, pl.ds(cc, _LANES)],
                acc2 + mask_v[2, rr, pl.ds(cc, _LANES)],
                acc3 + mask_v[3, rr, pl.ds(cc, _LANES)])
    nacc = lax.fori_loop(0, _CHUNK // _LANES, n_body, (zero,) * _B)
    n_v[...] = nacc[0] + nacc[1] + nacc[2] + nacc[3]

    def group(gi, carry):
        del carry
        c0 = gi * _G
        acc = [zero] * (2 * _G)
        for b in range(_B):
            for h in range(2):
                slot = h
                pltpu.make_async_copy(
                    img.at[0, pl.ds(0, _G), pl.ds(0, _HROWS), :],
                    buf.at[slot], sems[slot]).wait()
                def step(i, a):
                    a = list(a)
                    r = lax.div(i * _UNROLL, _VPR)
                    cb = lax.rem(i * _UNROLL, _VPR) * _LANES
                    for u in range(_UNROLL):
                        m = mask_v[b, h * _HROWS + r,
                                   pl.ds(cb + u * _LANES, _LANES)]
                        for g in range(_G):
                            x = buf[slot, g, r,
                                    pl.ds(cb + u * _LANES, _LANES)]
                            xm = x * m
                            a[2 * g] = a[2 * g] + xm
                            a[2 * g + 1] = a[2 * g + 1] + xm * xm
                    return tuple(a)

                acc = list(lax.fori_loop(0, _STEPS, step, tuple(acc)))

                if b < _B - 1:
                    issue(b + 1, h, c0, slot)
                else:
                    @pl.when(gi + 1 < _NG)
                    def _():
                        issue(0, h, c0 + _G, slot)

        for g in range(_G):
            s_v[c0 + g] = acc[2 * g]
            ss_v[c0 + g] = acc[2 * g + 1]
        return 0

    lax.fori_loop(0, _NG, group, 0)

    pltpu.sync_copy(s_v, out_s.at[wid])
    pltpu.sync_copy(ss_v, out_ss.at[wid])
    pltpu.sync_copy(n_v, out_n.at[wid])


def _tc_partials_body(img_ref, mask_ref, out_ref):
    b = pl.program_id(0)
    cg = pl.program_id(1)

    @pl.when((b == 0) & (cg == 0))
    def _():
        out_ref[...] = jnp.zeros_like(out_ref)

    x = img_ref[0]
    m = mask_ref[0]
    xm = x * m
    s4 = jnp.sum(xm.reshape(_CGRP, _H // 8, 8, _W), axis=1)
    ss4 = jnp.sum((xm * xm).reshape(_CGRP, _H // 8, 8, _W), axis=1)
    s2 = jnp.sum(s4.reshape(_CGRP, 8, _W // 128, 128), axis=2)
    ss2 = jnp.sum(ss4.reshape(_CGRP, 8, _W // 128, 128), axis=2)
    out_ref[0, cg] += s2
    out_ref[1, cg] += ss2


def _finalize_body(s_ref, ss_ref, n_ref, tc_ref, out_ref):
    n = jnp.sum(n_ref[...])
    s_sc = jnp.sum(s_ref[...], axis=(0, 2))
    ss_sc = jnp.sum(ss_ref[...], axis=(0, 2))
    var_sc = (ss_sc - s_sc * s_sc / n) / (n - 1.0)
    s_tc = jnp.sum(jnp.sum(tc_ref[0], axis=3), axis=2)
    ss_tc = jnp.sum(jnp.sum(tc_ref[1], axis=3), axis=2)
    var_tc = (ss_tc - s_tc * s_tc / n) / (n - 1.0)
    total = jnp.sum(jnp.sqrt(var_sc)) + jnp.sum(jnp.sqrt(var_tc))
    out_ref[...] = (total / _C).reshape(1, 1)


@jax.jit
def kernel(img, mask):
    mask_f4 = mask.astype(jnp.float32)

    mesh = plsc.VectorSubcoreMesh(core_axis_name="c", subcore_axis_name="s")
    sc_partials = pl.kernel(
        _sc_partials_body,
        out_type=(
            jax.ShapeDtypeStruct((_NW, _C_SC, _LANES), jnp.float32),
            jax.ShapeDtypeStruct((_NW, _C_SC, _LANES), jnp.float32),
            jax.ShapeDtypeStruct((_NW, _LANES), jnp.float32),
        ),
        mesh=mesh,
        scratch_types=[
            pltpu.VMEM((_B, _ROWS, _W), jnp.float32),
            pltpu.VMEM((2, _G, _HROWS, _W), jnp.float32),
            pltpu.VMEM((_C_SC, _LANES), jnp.float32),
            pltpu.VMEM((_C_SC, _LANES), jnp.float32),
            pltpu.VMEM((_LANES,), jnp.float32),
            pltpu.SemaphoreType.DMA,
            pltpu.SemaphoreType.DMA,
        ],
    )
    part_s, part_ss, part_n = sc_partials(img, mask_f4)

    tc_part = pl.pallas_call(
        _tc_partials_body,
        grid=(_B, _NCG),
        in_specs=[
            pl.BlockSpec((1, _CGRP, _H, _W),
                         lambda b, cg: (b, cg + _C_SC // _CGRP, 0, 0)),
            pl.BlockSpec((1, 1, _H, _W),
                         lambda b, cg: (b, 0, 0, 0)),
        ],
        out_specs=pl.BlockSpec((2, _NCG, _CGRP, 8, 128),
                               lambda b, cg: (0, 0, 0, 0, 0)),
        out_shape=jax.ShapeDtypeStruct((2, _NCG, _CGRP, 8, 128), jnp.float32),
        compiler_params=pltpu.CompilerParams(
            dimension_semantics=("arbitrary", "arbitrary"),
        ),
    )(img, mask_f4)

    loss = pl.pallas_call(
        _finalize_body,
        out_shape=jax.ShapeDtypeStruct((1, 1), jnp.float32),
    )(part_s, part_ss, part_n, tc_part)
    return loss[0, 0]
